# baseline (speedup 1.0000x reference)
import functools

import jax
import jax.numpy as jnp
from jax import lax
from jax.experimental import pallas as pl
from jax.experimental.pallas import tpu as pltpu
from jax.experimental.pallas import tpu_sc as plsc

NUM_LOCS = 100000
POI_DIM = 32
BATCH = 4096
HIST_LEN = 200

NC = 2
NS = 16
NW = NC * NS

BPW = BATCH // NW
CHUNKS = ((0, 128), (128, 72))
NB = 2
NGRP = BPW // NB
NBUF = 4

_mesh = plsc.VectorSubcoreMesh(core_axis_name="c", subcore_axis_name="s")


@functools.partial(
    pl.kernel,
    mesh=_mesh,
    compiler_params=pltpu.CompilerParams(use_tc_tiling_on_sc=False),
    out_type=jax.ShapeDtypeStruct((BATCH, HIST_LEN, POI_DIM), jnp.float32),
    name="poi_gather",
    scratch_types=[
        pltpu.VMEM((BPW, HIST_LEN), jnp.int32),
        pltpu.VMEM((NBUF, NB, HIST_LEN, POI_DIM), jnp.float32),
        pltpu.SemaphoreType.DMA,
        pltpu.SemaphoreType.DMA,
    ],
)
def _poi_gather(x_hbm, tab_hbm, out_hbm, idx_v, rows_v, gsem, osem):
    wid = lax.axis_index("s") * NC + lax.axis_index("c")
    wbase = wid * BPW
    pltpu.sync_copy(x_hbm.at[pl.ds(wbase, BPW)], idx_v)

    def fire_gathers(g, b):
        for j in range(NB):
            for off, sz in CHUNKS:
                pltpu.async_copy(
                    tab_hbm.at[idx_v.at[g * NB + j, pl.ds(off, sz)]],
                    rows_v.at[b, j, pl.ds(off, sz)],
                    gsem,
                )

    def wait_gathers(b):
        pltpu.make_async_copy(out_hbm.at[pl.ds(0, NB)], rows_v.at[b], gsem).wait()

    def wait_out(b):
        pltpu.make_async_copy(out_hbm.at[pl.ds(0, NB)], rows_v.at[b], osem).wait()

    fire_gathers(0, 0)
    fire_gathers(1, 1)

    def body(gg, carry):
        for phase in range(NBUF):
            g = gg * NBUF + phase
            b = phase
            wait_gathers(b)
            pltpu.async_copy(rows_v.at[b], out_hbm.at[pl.ds(wbase + g * NB, NB)], osem)

            @pl.when(g >= 2)
            def _():
                wait_out((phase + 2) % NBUF)

            @pl.when(g + 2 < NGRP)
            def _():
                fire_gathers(g + 2, (phase + 2) % NBUF)

        return carry

    lax.fori_loop(0, NGRP // NBUF, body, 0)
    wait_out(NBUF - 2)
    wait_out(NBUF - 1)


def kernel(x, POI):
    return _poi_gather(x.astype(jnp.int32), POI)


# speedup vs baseline: 1.0016x; 1.0016x over previous
import functools

import jax
import jax.numpy as jnp
from jax import lax
from jax.experimental import pallas as pl
from jax.experimental.pallas import tpu as pltpu
from jax.experimental.pallas import tpu_sc as plsc

NUM_LOCS = 100000
POI_DIM = 32
BATCH = 4096
HIST_LEN = 200

NC = 2
NS = 16
NW = NC * NS

BPW = BATCH // NW
CHUNKS = ((0, 128), (128, 72))
NB = 2
NGRP = BPW // NB
NBUF = 4

_mesh = plsc.VectorSubcoreMesh(core_axis_name="c", subcore_axis_name="s")


@functools.partial(
    pl.kernel,
    mesh=_mesh,
    compiler_params=pltpu.CompilerParams(use_tc_tiling_on_sc=False),
    out_type=jax.ShapeDtypeStruct((BATCH // NB, NB, HIST_LEN, POI_DIM), jnp.float32),
    name="poi_gather",
    scratch_types=[
        pltpu.VMEM((BPW, HIST_LEN), jnp.int32),
        pltpu.VMEM((NBUF, NB, HIST_LEN, POI_DIM), jnp.float32),
        pltpu.SemaphoreType.DMA,
        pltpu.SemaphoreType.DMA,
    ],
)
def _poi_gather(x_hbm, tab_hbm, out_hbm, idx_v, rows_v, gsem, osem):
    wid = lax.axis_index("s") * NC + lax.axis_index("c")
    wbase = wid * BPW
    wobase = wid * NGRP
    pltpu.sync_copy(x_hbm.at[pl.ds(wbase, BPW)], idx_v)

    def fire_gathers(g, b):
        for j in range(NB):
            for off, sz in CHUNKS:
                pltpu.async_copy(
                    tab_hbm.at[idx_v.at[g * NB + j, pl.ds(off, sz)]],
                    rows_v.at[b, j, pl.ds(off, sz)],
                    gsem,
                )

    def wait_gathers(b):
        pltpu.make_async_copy(out_hbm.at[0], rows_v.at[b], gsem).wait()

    def wait_out(b):
        pltpu.make_async_copy(out_hbm.at[0], rows_v.at[b], osem).wait()

    fire_gathers(0, 0)
    fire_gathers(1, 1)

    def body(gg, carry):
        for phase in range(NBUF):
            g = gg * NBUF + phase
            b = phase
            wait_gathers(b)
            pltpu.async_copy(rows_v.at[b], out_hbm.at[wobase + g], osem)

            @pl.when(g >= 2)
            def _():
                wait_out((phase + 2) % NBUF)

            @pl.when(g + 2 < NGRP)
            def _():
                fire_gathers(g + 2, (phase + 2) % NBUF)

        return carry

    lax.fori_loop(0, NGRP // NBUF, body, 0)
    wait_out(NBUF - 2)
    wait_out(NBUF - 1)


def kernel(x, POI):
    out = _poi_gather(x.astype(jnp.int32), POI)
    return out.reshape(BATCH, HIST_LEN, POI_DIM)


# speedup vs baseline: 1.0058x; 1.0042x over previous
import functools

import jax
import jax.numpy as jnp
from jax import lax
from jax.experimental import pallas as pl
from jax.experimental.pallas import tpu as pltpu
from jax.experimental.pallas import tpu_sc as plsc

NUM_LOCS = 100000
POI_DIM = 32
BATCH = 4096
HIST_LEN = 200

NC = 2
NS = 16
NW = NC * NS

NIDX = BATCH * HIST_LEN // NW
CH = 128
G = 4
GROUP = G * CH
NGRP = NIDX // GROUP
K = 3
NBUF = 2 * K

_mesh = plsc.VectorSubcoreMesh(core_axis_name="c", subcore_axis_name="s")


@functools.partial(
    pl.kernel,
    mesh=_mesh,
    compiler_params=pltpu.CompilerParams(use_tc_tiling_on_sc=False),
    out_type=jax.ShapeDtypeStruct((NW * NGRP, GROUP, POI_DIM), jnp.float32),
    name="poi_gather",
    scratch_types=[
        pltpu.VMEM((NIDX,), jnp.int32),
        pltpu.VMEM((NBUF, GROUP, POI_DIM), jnp.float32),
        pltpu.SemaphoreType.DMA,
        pltpu.SemaphoreType.DMA,
    ],
)
def _poi_gather(x_hbm, tab_hbm, out_hbm, idx_v, rows_v, gsem, osem):
    wid = lax.axis_index("s") * NC + lax.axis_index("c")
    wobase = wid * NGRP
    pltpu.sync_copy(x_hbm.at[pl.ds(wid * NIDX, NIDX)], idx_v)

    def fire_gathers(g, b):
        for j in range(G):
            pltpu.async_copy(
                tab_hbm.at[idx_v.at[pl.ds(g * GROUP + j * CH, CH)]],
                rows_v.at[b, pl.ds(j * CH, CH)],
                gsem,
            )

    def wait_gathers(b):
        pltpu.make_async_copy(out_hbm.at[0], rows_v.at[b], gsem).wait()

    def wait_out(b):
        pltpu.make_async_copy(out_hbm.at[0], rows_v.at[b], osem).wait()

    for i in range(K):
        fire_gathers(i, i)

    def body(g, carry):
        b = lax.rem(g, NBUF)
        bn = lax.rem(g + K, NBUF)
        wait_gathers(b)
        pltpu.async_copy(rows_v.at[b], out_hbm.at[wobase + g], osem)

        @pl.when(g >= K)
        def _():
            wait_out(bn)

        @pl.when(g + K < NGRP)
        def _():
            fire_gathers(g + K, bn)

        return carry

    lax.fori_loop(0, NGRP, body, 0)
    for i in range(K):
        wait_out(lax.rem(NGRP - K + i, NBUF))


def kernel(x, POI):
    out = _poi_gather(x.reshape(-1).astype(jnp.int32), POI)
    return out.reshape(BATCH, HIST_LEN, POI_DIM)
